# baseline (speedup 1.0000x reference)
import functools

import jax
import jax.numpy as jnp
from jax import lax
from jax.experimental import pallas as pl
from jax.experimental.pallas import tpu as pltpu
from jax.experimental.pallas import tpu_sc as plsc

D = 1024
H = 4096
E = 8
K = 2

BLK = 256
NBLK = 40
P = NBLK * BLK
HT = 512
NHT = H // HT

SC_NC = 2
SC_NS = 16
NW = SC_NC * SC_NS

DISP_CH = 40
DISP_NCH = P // NW // DISP_CH


def _gating_body(x_ref, wg_ref, iout_ref, wout_ref):
    logits = jax.lax.dot_general(
        x_ref[...], wg_ref[...], (((1,), (1,)), ((), ())),
        preferred_element_type=jnp.float32)
    m = jnp.max(logits, axis=1, keepdims=True)
    ex = jnp.exp(logits - m)
    p = ex / jnp.sum(ex, axis=1, keepdims=True)
    i1 = jnp.argmax(p, axis=1)
    p1 = jnp.max(p, axis=1)
    cols = lax.broadcasted_iota(jnp.int32, p.shape, 1)
    p_masked = jnp.where(cols == i1[:, None], -jnp.inf, p)
    i2 = jnp.argmax(p_masked, axis=1)
    p2 = jnp.max(p_masked, axis=1)
    s = p1 + p2 + 1e-6
    iout_ref[0, :] = i1.astype(jnp.int32)
    iout_ref[1, :] = i2.astype(jnp.int32)
    wout_ref[0, :] = p1 / s
    wout_ref[1, :] = p2 / s


def _gating(xf, Wg):
    n = xf.shape[0]
    rows = 1024
    grid = (n // rows,)
    iout, wout = pl.pallas_call(
        _gating_body,
        grid=grid,
        in_specs=[
            pl.BlockSpec((rows, D), lambda i: (i, 0)),
            pl.BlockSpec((E, D), lambda i: (0, 0)),
        ],
        out_specs=[
            pl.BlockSpec((2, rows), lambda i: (0, i)),
            pl.BlockSpec((2, rows), lambda i: (0, i)),
        ],
        out_shape=[
            jax.ShapeDtypeStruct((2, n), jnp.int32),
            jax.ShapeDtypeStruct((2, n), jnp.float32),
        ],
    )(xf, Wg)
    return iout, wout


def _dispatch_body(xf_hbm, src_hbm, xs_hbm, idx_v, buf_v, sem):
    wid = lax.axis_index("s") * SC_NC + lax.axis_index("c")
    base = wid * (DISP_NCH * DISP_CH)
    pltpu.sync_copy(src_hbm.at[wid], idx_v)
    for c in range(DISP_NCH):
        pltpu.async_copy(xf_hbm.at[idx_v.at[c]], buf_v, sem).wait()
        pltpu.sync_copy(buf_v, xs_hbm.at[pl.ds(base + c * DISP_CH, DISP_CH)])


def _dispatch(xf, src3):
    mesh = plsc.VectorSubcoreMesh(core_axis_name="c", subcore_axis_name="s")
    return pl.kernel(
        _dispatch_body,
        out_type=jax.ShapeDtypeStruct((P, D), jnp.float32),
        mesh=mesh,
        scratch_types=[
            pltpu.VMEM((DISP_NCH, DISP_CH), jnp.int32),
            pltpu.VMEM((DISP_CH, D), jnp.float32),
            pltpu.SemaphoreType.DMA,
        ],
    )(xf, src3)


def _ffn_body(eid_ref, xs_ref, w1_ref, b1_ref, w2_ref, b2_ref, wrep_ref,
              y_ref):
    ht = pl.program_id(1)

    @pl.when(ht == 0)
    def _init():
        y_ref[...] = jnp.broadcast_to(b2_ref[0, 0, :], (BLK, D))

    h = jax.lax.dot_general(
        xs_ref[...].astype(jnp.bfloat16), w1_ref[0], (((1,), (1,)), ((), ())),
        preferred_element_type=jnp.float32)
    h = h + b1_ref[0, 0, :]
    h = 0.5 * h * (1.0 + lax.erf(h * 0.7071067811865476))
    y_ref[...] += jax.lax.dot_general(
        h.astype(jnp.bfloat16), w2_ref[0], (((1,), (1,)), ((), ())),
        preferred_element_type=jnp.float32)

    @pl.when(ht == NHT - 1)
    def _scale():
        y_ref[...] *= wrep_ref[:, :1]


def _ffn(block_eid, xs, W1, b1, W2, b2, wrep):
    b1r = b1.reshape(E, 1, H)
    b2r = b2.reshape(E, 1, D)
    return pl.pallas_call(
        _ffn_body,
        grid_spec=pltpu.PrefetchScalarGridSpec(
            num_scalar_prefetch=1,
            grid=(NBLK, NHT),
            in_specs=[
                pl.BlockSpec((BLK, D), lambda g, t, eid: (g, 0)),
                pl.BlockSpec((1, HT, D), lambda g, t, eid: (eid[g], t, 0)),
                pl.BlockSpec((1, 1, HT), lambda g, t, eid: (eid[g], 0, t)),
                pl.BlockSpec((1, D, HT), lambda g, t, eid: (eid[g], 0, t)),
                pl.BlockSpec((1, 1, D), lambda g, t, eid: (eid[g], 0, 0)),
                pl.BlockSpec((BLK, 128), lambda g, t, eid: (g, 0)),
            ],
            out_specs=pl.BlockSpec((BLK, D), lambda g, t, eid: (g, 0)),
        ),
        out_shape=jax.ShapeDtypeStruct((P, D), jnp.float32),
        compiler_params=pltpu.CompilerParams(
            dimension_semantics=("arbitrary", "arbitrary")),
    )(block_eid, xs, W1, b1r, W2, b2r, wrep)


CB_CH = 16
CB_NCH = 4096 // NW // CB_CH


def _combine_body(y_hbm, p0_hbm, p1_hbm, out_hbm, i0_v, i1_v, b0_v, b1_v,
                  sem0, sem1):
    wid = lax.axis_index("s") * SC_NC + lax.axis_index("c")
    base = wid * (CB_NCH * CB_CH)
    pltpu.sync_copy(p0_hbm.at[wid], i0_v)
    pltpu.sync_copy(p1_hbm.at[wid], i1_v)
    for c in range(CB_NCH):
        cp0 = pltpu.async_copy(y_hbm.at[i0_v.at[c]], b0_v, sem0)
        cp1 = pltpu.async_copy(y_hbm.at[i1_v.at[c]], b1_v, sem1)
        cp0.wait()
        cp1.wait()

        def row(r, _):
            def vec(v, __):
                sl = pl.ds(v * 16, 16)
                b0_v[r, sl] = b0_v[r, sl] + b1_v[r, sl]
                return __
            return lax.fori_loop(0, D // 16, vec, 0)

        lax.fori_loop(0, CB_CH, row, 0)
        pltpu.sync_copy(b0_v, out_hbm.at[pl.ds(base + c * CB_CH, CB_CH)])


def _combine(y, pos0r, pos1r):
    mesh = plsc.VectorSubcoreMesh(core_axis_name="c", subcore_axis_name="s")
    return pl.kernel(
        _combine_body,
        out_type=jax.ShapeDtypeStruct((4096, D), jnp.float32),
        mesh=mesh,
        scratch_types=[
            pltpu.VMEM((CB_NCH, CB_CH), jnp.int32),
            pltpu.VMEM((CB_NCH, CB_CH), jnp.int32),
            pltpu.VMEM((CB_CH, D), jnp.float32),
            pltpu.VMEM((CB_CH, D), jnp.float32),
            pltpu.SemaphoreType.DMA,
            pltpu.SemaphoreType.DMA,
        ],
    )(y, pos0r, pos1r)


def kernel(x, Wg, W1, b1, W2, b2):
    Bq, Lq, Dq = x.shape
    n = Bq * Lq
    xf = x.reshape(n, Dq)

    iout, wout = _gating(xf, Wg)
    e_pair = jnp.stack([iout[0], iout[1]], axis=1).reshape(-1)
    w_pair = jnp.stack([wout[0], wout[1]], axis=1).reshape(-1)

    oh = (e_pair[:, None] == jnp.arange(E, dtype=jnp.int32)[None, :]
          ).astype(jnp.int32)
    cum = jnp.cumsum(oh, axis=0)
    rank = jnp.take_along_axis(cum, e_pair[:, None], axis=1)[:, 0] - 1
    counts = cum[-1]
    nblk = (counts + BLK - 1) // BLK
    blk_cum = jnp.cumsum(nblk)
    blk_start = blk_cum - nblk
    dest = blk_start[e_pair] * BLK + rank

    tok = jnp.arange(2 * n, dtype=jnp.int32) // K
    src = jnp.zeros((P,), jnp.int32).at[dest].set(tok)
    wsr = jnp.zeros((P,), jnp.float32).at[dest].set(w_pair)
    wrep = jnp.broadcast_to(wsr[:, None], (P, 128))

    gids = jnp.arange(NBLK, dtype=jnp.int32)
    block_eid = jnp.minimum(
        jnp.searchsorted(blk_cum, gids, side="right"), E - 1).astype(jnp.int32)

    src3 = src.reshape(NW, DISP_NCH, DISP_CH)
    xs = _dispatch(xf, src3)

    y = _ffn(block_eid, xs, W1.astype(jnp.bfloat16), b1,
             W2.astype(jnp.bfloat16), b2, wrep)

    destm = dest.reshape(n, K)
    pos0r = destm[:, 0].reshape(NW, CB_NCH, CB_CH).astype(jnp.int32)
    pos1r = destm[:, 1].reshape(NW, CB_NCH, CB_CH).astype(jnp.int32)
    out = _combine(y, pos0r, pos1r)
    return out.reshape(Bq, Lq, Dq)


# speedup vs baseline: 1.3422x; 1.3422x over previous
import functools

import jax
import jax.numpy as jnp
from jax import lax
from jax.experimental import pallas as pl
from jax.experimental.pallas import tpu as pltpu
from jax.experimental.pallas import tpu_sc as plsc

D = 1024
H = 4096
E = 8
K = 2

BLK = 256
NBLK = 40
P = NBLK * BLK
HT = 512
NHT = H // HT

SC_NC = 2
SC_NS = 16
NW = SC_NC * SC_NS

DISP_CH = 40
DISP_NCH = P // NW // DISP_CH


def _gating_body(x_ref, wg_ref, iout_ref, wout_ref):
    logits = jax.lax.dot_general(
        x_ref[...], wg_ref[...], (((1,), (1,)), ((), ())),
        preferred_element_type=jnp.float32)
    m = jnp.max(logits, axis=1, keepdims=True)
    ex = jnp.exp(logits - m)
    p = ex / jnp.sum(ex, axis=1, keepdims=True)
    i1 = jnp.argmax(p, axis=1)
    p1 = jnp.max(p, axis=1)
    cols = lax.broadcasted_iota(jnp.int32, p.shape, 1)
    p_masked = jnp.where(cols == i1[:, None], -jnp.inf, p)
    i2 = jnp.argmax(p_masked, axis=1)
    p2 = jnp.max(p_masked, axis=1)
    s = p1 + p2 + 1e-6
    iout_ref[0, :] = i1.astype(jnp.int32)
    iout_ref[1, :] = i2.astype(jnp.int32)
    wout_ref[0, :] = p1 / s
    wout_ref[1, :] = p2 / s


def _gating(xf, Wg):
    n = xf.shape[0]
    rows = 1024
    grid = (n // rows,)
    iout, wout = pl.pallas_call(
        _gating_body,
        grid=grid,
        in_specs=[
            pl.BlockSpec((rows, D), lambda i: (i, 0)),
            pl.BlockSpec((E, D), lambda i: (0, 0)),
        ],
        out_specs=[
            pl.BlockSpec((2, rows), lambda i: (0, i)),
            pl.BlockSpec((2, rows), lambda i: (0, i)),
        ],
        out_shape=[
            jax.ShapeDtypeStruct((2, n), jnp.int32),
            jax.ShapeDtypeStruct((2, n), jnp.float32),
        ],
    )(xf, Wg)
    return iout, wout


def _dispatch_body(xf_hbm, src_hbm, xs_hbm, idx_v, buf_v, sem):
    wid = lax.axis_index("s") * SC_NC + lax.axis_index("c")
    base = wid * (DISP_NCH * DISP_CH)
    pltpu.sync_copy(src_hbm.at[wid], idx_v)
    for c in range(DISP_NCH):
        pltpu.async_copy(xf_hbm.at[idx_v.at[c]], buf_v, sem).wait()
        pltpu.sync_copy(buf_v, xs_hbm.at[pl.ds(base + c * DISP_CH, DISP_CH)])


def _dispatch(xf, src3):
    mesh = plsc.VectorSubcoreMesh(core_axis_name="c", subcore_axis_name="s")
    return pl.kernel(
        _dispatch_body,
        out_type=jax.ShapeDtypeStruct((P, D), jnp.float32),
        mesh=mesh,
        scratch_types=[
            pltpu.VMEM((DISP_NCH, DISP_CH), jnp.int32),
            pltpu.VMEM((DISP_CH, D), jnp.float32),
            pltpu.SemaphoreType.DMA,
        ],
    )(xf, src3)


def _ffn_body(eid_ref, fg_ref, tot_ref, xs_ref, w1_ref, b1_ref, w2_ref,
              b2_ref, wrep_ref, y_ref, w1s, w2s, xbs):
    g = pl.program_id(0)
    t = pl.program_id(1)
    lw = (g == fg_ref[g]) & (g < tot_ref[0])

    @pl.when(lw)
    def _load():
        sl = pl.ds(t * HT, HT)
        w1s[sl, :] = w1_ref[0].astype(jnp.bfloat16)
        w2s[:, sl] = w2_ref[0].astype(jnp.bfloat16)

    @pl.when(g < tot_ref[0])
    def _compute():
        sl = pl.ds(t * HT, HT)

        @pl.when(t == 0)
        def _init():
            xbs[...] = xs_ref[...].astype(jnp.bfloat16)
            y_ref[...] = jnp.broadcast_to(b2_ref[0, 0, :], (BLK, D))

        h = jax.lax.dot_general(
            xbs[...], w1s[sl, :], (((1,), (1,)), ((), ())),
            preferred_element_type=jnp.float32)
        h = h + b1_ref[0, 0, sl]
        h = 0.5 * h * (1.0 + lax.erf(h * 0.7071067811865476))
        y_ref[...] += jax.lax.dot_general(
            h.astype(jnp.bfloat16), w2s[:, sl], (((1,), (1,)), ((), ())),
            preferred_element_type=jnp.float32)

        @pl.when(t == NHT - 1)
        def _scale():
            y_ref[...] *= wrep_ref[:, :1]


def _ffn(block_eid, first_g, tot, xs, W1, b1, W2, b2, wrep):
    b1r = b1.reshape(E, 1, H)
    b2r = b2.reshape(E, 1, D)

    def _wsel(g, t, eid, fg, tot_):
        return jnp.where((g == fg[g]) & (g < tot_[0]), t, NHT - 1)

    return pl.pallas_call(
        _ffn_body,
        grid_spec=pltpu.PrefetchScalarGridSpec(
            num_scalar_prefetch=3,
            grid=(NBLK, NHT),
            in_specs=[
                pl.BlockSpec((BLK, D), lambda g, t, eid, fg, tot_: (g, 0)),
                pl.BlockSpec((1, HT, D),
                             lambda g, t, eid, fg, tot_:
                             (eid[g], _wsel(g, t, eid, fg, tot_), 0)),
                pl.BlockSpec((1, 1, H),
                             lambda g, t, eid, fg, tot_: (eid[g], 0, 0)),
                pl.BlockSpec((1, D, HT),
                             lambda g, t, eid, fg, tot_:
                             (eid[g], 0, _wsel(g, t, eid, fg, tot_))),
                pl.BlockSpec((1, 1, D),
                             lambda g, t, eid, fg, tot_: (eid[g], 0, 0)),
                pl.BlockSpec((BLK, 128), lambda g, t, eid, fg, tot_: (g, 0)),
            ],
            out_specs=pl.BlockSpec((BLK, D), lambda g, t, eid, fg, tot_: (g, 0)),
            scratch_shapes=[
                pltpu.VMEM((H, D), jnp.bfloat16),
                pltpu.VMEM((D, H), jnp.bfloat16),
                pltpu.VMEM((BLK, D), jnp.bfloat16),
            ],
        ),
        out_shape=jax.ShapeDtypeStruct((P, D), jnp.float32),
        compiler_params=pltpu.CompilerParams(
            dimension_semantics=("arbitrary", "arbitrary")),
    )(block_eid, first_g, tot, xs, W1, b1r, W2, b2r, wrep)


CB_CH = 16
CB_NCH = 4096 // NW // CB_CH


def _combine_body(y_hbm, p0_hbm, p1_hbm, out_hbm, i0_v, i1_v, b0_v, b1_v,
                  sem0, sem1):
    wid = lax.axis_index("s") * SC_NC + lax.axis_index("c")
    base = wid * (CB_NCH * CB_CH)
    pltpu.sync_copy(p0_hbm.at[wid], i0_v)
    pltpu.sync_copy(p1_hbm.at[wid], i1_v)
    for c in range(CB_NCH):
        cp0 = pltpu.async_copy(y_hbm.at[i0_v.at[c]], b0_v, sem0)
        cp1 = pltpu.async_copy(y_hbm.at[i1_v.at[c]], b1_v, sem1)
        cp0.wait()
        cp1.wait()

        def row(r, _):
            def vec(v, __):
                sl = pl.ds(v * 16, 16)
                b0_v[r, sl] = b0_v[r, sl] + b1_v[r, sl]
                return __
            return lax.fori_loop(0, D // 16, vec, 0)

        lax.fori_loop(0, CB_CH, row, 0)
        pltpu.sync_copy(b0_v, out_hbm.at[pl.ds(base + c * CB_CH, CB_CH)])


def _combine(y, pos0r, pos1r):
    mesh = plsc.VectorSubcoreMesh(core_axis_name="c", subcore_axis_name="s")
    return pl.kernel(
        _combine_body,
        out_type=jax.ShapeDtypeStruct((4096, D), jnp.float32),
        mesh=mesh,
        scratch_types=[
            pltpu.VMEM((CB_NCH, CB_CH), jnp.int32),
            pltpu.VMEM((CB_NCH, CB_CH), jnp.int32),
            pltpu.VMEM((CB_CH, D), jnp.float32),
            pltpu.VMEM((CB_CH, D), jnp.float32),
            pltpu.SemaphoreType.DMA,
            pltpu.SemaphoreType.DMA,
        ],
    )(y, pos0r, pos1r)


def kernel(x, Wg, W1, b1, W2, b2):
    Bq, Lq, Dq = x.shape
    n = Bq * Lq
    xf = x.reshape(n, Dq)

    iout, wout = _gating(xf, Wg)
    e_pair = jnp.stack([iout[0], iout[1]], axis=1).reshape(-1)
    w_pair = jnp.stack([wout[0], wout[1]], axis=1).reshape(-1)

    oh = (e_pair[:, None] == jnp.arange(E, dtype=jnp.int32)[None, :]
          ).astype(jnp.int32)
    cum = jnp.cumsum(oh, axis=0)
    rank = jnp.take_along_axis(cum, e_pair[:, None], axis=1)[:, 0] - 1
    counts = cum[-1]
    nblk = (counts + BLK - 1) // BLK
    blk_cum = jnp.cumsum(nblk)
    blk_start = blk_cum - nblk
    dest = blk_start[e_pair] * BLK + rank

    tok = jnp.arange(2 * n, dtype=jnp.int32) // K
    src = jnp.zeros((P,), jnp.int32).at[dest].set(tok)
    wsr = jnp.zeros((P,), jnp.float32).at[dest].set(w_pair)
    wrep = jnp.broadcast_to(wsr[:, None], (P, 128))

    gids = jnp.arange(NBLK, dtype=jnp.int32)
    block_eid = jnp.minimum(
        jnp.searchsorted(blk_cum, gids, side="right"), E - 1).astype(jnp.int32)

    src3 = src.reshape(NW, DISP_NCH, DISP_CH)
    xs = _dispatch(xf, src3)

    first_g = blk_start[block_eid].astype(jnp.int32)
    tot = blk_cum[E - 1:].astype(jnp.int32)
    y = _ffn(block_eid, first_g, tot, xs, W1, b1, W2, b2, wrep)

    destm = dest.reshape(n, K)
    pos0r = destm[:, 0].reshape(NW, CB_NCH, CB_CH).astype(jnp.int32)
    pos1r = destm[:, 1].reshape(NW, CB_NCH, CB_CH).astype(jnp.int32)
    out = _combine(y, pos0r, pos1r)
    return out.reshape(Bq, Lq, Dq)


# speedup vs baseline: 1.4152x; 1.0543x over previous
import functools

import jax
import jax.numpy as jnp
from jax import lax
from jax.experimental import pallas as pl
from jax.experimental.pallas import tpu as pltpu
from jax.experimental.pallas import tpu_sc as plsc

D = 1024
H = 4096
E = 8
K = 2

BLK = 256
NBLK = 40
P = NBLK * BLK
HT = 512
NHT = H // HT

SC_NC = 2
SC_NS = 16
NW = SC_NC * SC_NS

DISP_CH = 40
DISP_NCH = P // NW // DISP_CH


def _gating_body(x_ref, wg_ref, iout_ref, wout_ref):
    logits = jax.lax.dot_general(
        x_ref[...], wg_ref[...], (((1,), (1,)), ((), ())),
        preferred_element_type=jnp.float32)
    m = jnp.max(logits, axis=1, keepdims=True)
    ex = jnp.exp(logits - m)
    p = ex / jnp.sum(ex, axis=1, keepdims=True)
    i1 = jnp.argmax(p, axis=1)
    p1 = jnp.max(p, axis=1)
    cols = lax.broadcasted_iota(jnp.int32, p.shape, 1)
    p_masked = jnp.where(cols == i1[:, None], -jnp.inf, p)
    i2 = jnp.argmax(p_masked, axis=1)
    p2 = jnp.max(p_masked, axis=1)
    s = p1 + p2 + 1e-6
    iout_ref[0, :] = i1.astype(jnp.int32)
    iout_ref[1, :] = i2.astype(jnp.int32)
    wout_ref[0, :] = p1 / s
    wout_ref[1, :] = p2 / s


def _gating(xf, Wg):
    n = xf.shape[0]
    rows = 1024
    grid = (n // rows,)
    iout, wout = pl.pallas_call(
        _gating_body,
        grid=grid,
        in_specs=[
            pl.BlockSpec((rows, D), lambda i: (i, 0)),
            pl.BlockSpec((E, D), lambda i: (0, 0)),
        ],
        out_specs=[
            pl.BlockSpec((2, rows), lambda i: (0, i)),
            pl.BlockSpec((2, rows), lambda i: (0, i)),
        ],
        out_shape=[
            jax.ShapeDtypeStruct((2, n), jnp.int32),
            jax.ShapeDtypeStruct((2, n), jnp.float32),
        ],
    )(xf, Wg)
    return iout, wout


def _dispatch_body(xf_hbm, src_hbm, xs_hbm, idx_v, buf0, buf1,
                   g0, g1, o0, o1):
    wid = lax.axis_index("s") * SC_NC + lax.axis_index("c")
    base = wid * (DISP_NCH * DISP_CH)
    pltpu.sync_copy(src_hbm.at[wid], idx_v)
    bufs = (buf0, buf1)
    gsems = (g0, g1)
    osems = (o0, o1)
    gcp = [None, None]
    ocp = [None, None]
    gcp[0] = pltpu.async_copy(xf_hbm.at[idx_v.at[0]], buf0, g0)
    for c in range(DISP_NCH):
        b = c & 1
        nb = b ^ 1
        gcp[b].wait()
        if c + 1 < DISP_NCH:
            if ocp[nb] is not None:
                ocp[nb].wait()
            gcp[nb] = pltpu.async_copy(
                xf_hbm.at[idx_v.at[c + 1]], bufs[nb], gsems[nb])
        ocp[b] = pltpu.async_copy(
            bufs[b], xs_hbm.at[pl.ds(base + c * DISP_CH, DISP_CH)], osems[b])
    ocp[0].wait()
    ocp[1].wait()


def _dispatch(xf, src3):
    mesh = plsc.VectorSubcoreMesh(core_axis_name="c", subcore_axis_name="s")
    return pl.kernel(
        _dispatch_body,
        out_type=jax.ShapeDtypeStruct((P, D), jnp.float32),
        mesh=mesh,
        scratch_types=[
            pltpu.VMEM((DISP_NCH, DISP_CH), jnp.int32),
            pltpu.VMEM((DISP_CH, D), jnp.float32),
            pltpu.VMEM((DISP_CH, D), jnp.float32),
            pltpu.SemaphoreType.DMA,
            pltpu.SemaphoreType.DMA,
            pltpu.SemaphoreType.DMA,
            pltpu.SemaphoreType.DMA,
        ],
    )(xf, src3)


def _ffn_body(eid_ref, fg_ref, tot_ref, xs_ref, w1_ref, b1_ref, w2_ref,
              b2_ref, wrep_ref, y_ref, w1s, w2s, xbs):
    g = pl.program_id(0)
    t = pl.program_id(1)
    lw = (g == fg_ref[g]) & (g < tot_ref[0])

    @pl.when(lw)
    def _load():
        sl = pl.ds(t * HT, HT)
        w1s[sl, :] = w1_ref[0].astype(jnp.bfloat16)
        w2s[:, sl] = w2_ref[0].astype(jnp.bfloat16)

    @pl.when(g < tot_ref[0])
    def _compute():
        sl = pl.ds(t * HT, HT)

        @pl.when(t == 0)
        def _init():
            xbs[...] = xs_ref[...].astype(jnp.bfloat16)
            y_ref[...] = jnp.broadcast_to(b2_ref[0, 0, :], (BLK, D))

        h = jax.lax.dot_general(
            xbs[...], w1s[sl, :], (((1,), (1,)), ((), ())),
            preferred_element_type=jnp.float32)
        h = h + b1_ref[0, 0, sl]
        h = 0.5 * h * (1.0 + lax.erf(h * 0.7071067811865476))
        y_ref[...] += jax.lax.dot_general(
            h.astype(jnp.bfloat16), w2s[:, sl], (((1,), (1,)), ((), ())),
            preferred_element_type=jnp.float32)

        @pl.when(t == NHT - 1)
        def _scale():
            y_ref[...] *= wrep_ref[:, :1]


def _ffn(block_eid, first_g, tot, xs, W1, b1, W2, b2, wrep):
    b1r = b1.reshape(E, 1, H)
    b2r = b2.reshape(E, 1, D)

    def _wsel(g, t, eid, fg, tot_):
        return jnp.where((g == fg[g]) & (g < tot_[0]), t, NHT - 1)

    return pl.pallas_call(
        _ffn_body,
        grid_spec=pltpu.PrefetchScalarGridSpec(
            num_scalar_prefetch=3,
            grid=(NBLK, NHT),
            in_specs=[
                pl.BlockSpec((BLK, D), lambda g, t, eid, fg, tot_: (g, 0)),
                pl.BlockSpec((1, HT, D),
                             lambda g, t, eid, fg, tot_:
                             (eid[g], _wsel(g, t, eid, fg, tot_), 0)),
                pl.BlockSpec((1, 1, H),
                             lambda g, t, eid, fg, tot_: (eid[g], 0, 0)),
                pl.BlockSpec((1, D, HT),
                             lambda g, t, eid, fg, tot_:
                             (eid[g], 0, _wsel(g, t, eid, fg, tot_))),
                pl.BlockSpec((1, 1, D),
                             lambda g, t, eid, fg, tot_: (eid[g], 0, 0)),
                pl.BlockSpec((BLK, 128), lambda g, t, eid, fg, tot_: (g, 0)),
            ],
            out_specs=pl.BlockSpec((BLK, D), lambda g, t, eid, fg, tot_: (g, 0)),
            scratch_shapes=[
                pltpu.VMEM((H, D), jnp.bfloat16),
                pltpu.VMEM((D, H), jnp.bfloat16),
                pltpu.VMEM((BLK, D), jnp.bfloat16),
            ],
        ),
        out_shape=jax.ShapeDtypeStruct((P, D), jnp.float32),
        compiler_params=pltpu.CompilerParams(
            dimension_semantics=("arbitrary", "arbitrary")),
    )(block_eid, first_g, tot, xs, W1, b1r, W2, b2r, wrep)


CB_CH = 16
CB_NCH = 4096 // NW // CB_CH


def _combine_body(y_hbm, pc_hbm, out_hbm, ic_v, buf0, buf1,
                  g0, g1, o0, o1):
    wid = lax.axis_index("s") * SC_NC + lax.axis_index("c")
    base = wid * (CB_NCH * CB_CH)
    pltpu.sync_copy(pc_hbm.at[wid], ic_v)
    bufs = (buf0, buf1)
    gsems = (g0, g1)
    osems = (o0, o1)
    gcp = [None, None]
    ocp = [None, None]
    gcp[0] = pltpu.async_copy(y_hbm.at[ic_v.at[0]], buf0, g0)
    for c in range(CB_NCH):
        b = c & 1
        nb = b ^ 1
        gcp[b].wait()
        if c + 1 < CB_NCH:
            if ocp[nb] is not None:
                ocp[nb].wait()
            gcp[nb] = pltpu.async_copy(
                y_hbm.at[ic_v.at[c + 1]], bufs[nb], gsems[nb])
        buf = bufs[b]
        for r in range(CB_CH):
            @plsc.parallel_loop(0, D // 16, 1, unroll=4)
            def _v(v, buf=buf, r=r):
                sl = pl.ds(v * 16, 16)
                buf[r, sl] = buf[r, sl] + buf[r + CB_CH, sl]
        ocp[b] = pltpu.async_copy(
            buf.at[pl.ds(0, CB_CH)],
            out_hbm.at[pl.ds(base + c * CB_CH, CB_CH)], osems[b])
    ocp[0].wait()
    ocp[1].wait()


def _combine(y, pcat):
    mesh = plsc.VectorSubcoreMesh(core_axis_name="c", subcore_axis_name="s")
    return pl.kernel(
        _combine_body,
        out_type=jax.ShapeDtypeStruct((4096, D), jnp.float32),
        mesh=mesh,
        scratch_types=[
            pltpu.VMEM((CB_NCH, 2 * CB_CH), jnp.int32),
            pltpu.VMEM((2 * CB_CH, D), jnp.float32),
            pltpu.VMEM((2 * CB_CH, D), jnp.float32),
            pltpu.SemaphoreType.DMA,
            pltpu.SemaphoreType.DMA,
            pltpu.SemaphoreType.DMA,
            pltpu.SemaphoreType.DMA,
        ],
    )(y, pcat)


def kernel(x, Wg, W1, b1, W2, b2):
    Bq, Lq, Dq = x.shape
    n = Bq * Lq
    xf = x.reshape(n, Dq)

    iout, wout = _gating(xf, Wg)
    e_pair = jnp.stack([iout[0], iout[1]], axis=1).reshape(-1)
    w_pair = jnp.stack([wout[0], wout[1]], axis=1).reshape(-1)

    oh = (e_pair[:, None] == jnp.arange(E, dtype=jnp.int32)[None, :]
          ).astype(jnp.int32)
    cum = jnp.cumsum(oh, axis=0)
    rank = jnp.take_along_axis(cum, e_pair[:, None], axis=1)[:, 0] - 1
    counts = cum[-1]
    nblk = (counts + BLK - 1) // BLK
    blk_cum = jnp.cumsum(nblk)
    blk_start = blk_cum - nblk
    dest = blk_start[e_pair] * BLK + rank

    tok = jnp.arange(2 * n, dtype=jnp.int32) // K
    src = jnp.zeros((P,), jnp.int32).at[dest].set(tok)
    wsr = jnp.zeros((P,), jnp.float32).at[dest].set(w_pair)
    wrep = jnp.broadcast_to(wsr[:, None], (P, 128))

    gids = jnp.arange(NBLK, dtype=jnp.int32)
    block_eid = jnp.minimum(
        jnp.searchsorted(blk_cum, gids, side="right"), E - 1).astype(jnp.int32)

    src3 = src.reshape(NW, DISP_NCH, DISP_CH)
    xs = _dispatch(xf, src3)

    first_g = blk_start[block_eid].astype(jnp.int32)
    tot = blk_cum[E - 1:].astype(jnp.int32)
    y = _ffn(block_eid, first_g, tot, xs, W1, b1, W2, b2, wrep)

    destm = dest.reshape(n, K)
    pcat = jnp.concatenate(
        [destm[:, 0].reshape(NW, CB_NCH, CB_CH),
         destm[:, 1].reshape(NW, CB_NCH, CB_CH)], axis=2).astype(jnp.int32)
    out = _combine(y, pcat)
    return out.reshape(Bq, Lq, Dq)
